# baseline (speedup 1.0000x reference)
import jax
import jax.numpy as jnp
from jax import lax
from jax.experimental import pallas as pl
from jax.experimental.pallas import tpu as pltpu
from jax.experimental.pallas import tpu_sc as plsc

_N = 1000
_F = 26
_NCOLS = _F * _N
_ROWBUF = _NCOLS + 16
_NW = 32
_NBUF = 4
_IDXW = 32


def _make_sc_kernel(b_per_w):
    nbuf = min(_NBUF, b_per_w)
    assert b_per_w % nbuf == 0
    mesh = plsc.VectorSubcoreMesh(core_axis_name="c", subcore_axis_name="s")

    def body(sh_hbm, out_hbm, idx_v, bufs, sems):
        wid = lax.axis_index("s") * 2 + lax.axis_index("c")
        base = wid * b_per_w
        pltpu.sync_copy(sh_hbm.at[pl.ds(base * _IDXW, b_per_w * _IDXW)],
                        idx_v)

        ones = jnp.full((16,), 1.0, jnp.float32)
        zeros = jnp.zeros((16,), jnp.float32)

        for k in range(nbuf):
            buf = bufs[k]

            @pl.loop(0, _ROWBUF // 16)
            def _(i):
                buf[pl.ds(i * 16, 16)] = zeros

        def scat(buf, row, val):
            i0 = idx_v[pl.ds(row * _IDXW, 16)]
            i1 = idx_v[pl.ds(row * _IDXW + 16, 16)]
            plsc.store_scatter(buf, [i0], val)
            plsc.store_scatter(buf, [i1], val)

        def fire(buf, sem, row):
            pltpu.async_copy(
                buf.at[pl.ds(0, _NCOLS)],
                out_hbm.at[pl.ds((base + row) * _NCOLS, _NCOLS)], sem)

        def wait(buf, sem, row):
            pltpu.make_async_copy(
                buf.at[pl.ds(0, _NCOLS)],
                out_hbm.at[pl.ds((base + row) * _NCOLS, _NCOLS)], sem).wait()

        for b in range(nbuf):
            scat(bufs[b], b, ones)
            fire(bufs[b], sems[b], b)

        @pl.loop(0, b_per_w // nbuf - 1)
        def _(it):
            done0 = it * nbuf
            for b in range(nbuf):
                wait(bufs[b], sems[b], done0 + b)
                scat(bufs[b], done0 + b, zeros)
                scat(bufs[b], done0 + nbuf + b, ones)
                fire(bufs[b], sems[b], done0 + nbuf + b)

        last0 = b_per_w - nbuf
        for b in range(nbuf):
            wait(bufs[b], sems[b], last0 + b)

    return pl.kernel(
        body,
        out_type=jax.ShapeDtypeStruct((b_per_w * _NW * _NCOLS,), jnp.float32),
        mesh=mesh,
        scratch_types=[
            pltpu.VMEM((b_per_w * _IDXW,), jnp.int32),
            [pltpu.VMEM((_ROWBUF,), jnp.float32) for _ in range(nbuf)],
            [pltpu.SemaphoreType.DMA for _ in range(nbuf)],
        ],
        compiler_params=pltpu.CompilerParams(needs_layout_passes=False),
    )


def kernel(x):
    b, f = x.shape
    assert f == _F

    shifted = x + (_N * jnp.arange(f, dtype=x.dtype))[None, :]
    sh = jnp.full((b, _IDXW), _NCOLS, jnp.int32).at[:, :f].set(shifted)

    bp = -(-b // _NW) * _NW
    if bp != b:
        sh = jnp.pad(sh, ((0, bp - b), (0, 0)), constant_values=_NCOLS)

    out = _make_sc_kernel(bp // _NW)(sh.reshape(-1))
    return out.reshape(bp, _NCOLS)[:b]
